# baseline (speedup 1.0000x reference)
import functools

import jax
import jax.numpy as jnp
import numpy as np
from jax import lax
from jax.experimental import pallas as pl
from jax.experimental.pallas import tpu as pltpu
from jax.experimental.pallas import tpu_sc as plsc

_IMG_SIZE = 224
_CENTERS_LEFT = [4, 1, 2, 24, 19, 16, 31, 31, 34, 34, 37, 43]
_CENTERS_RIGHT = [5, 8, 7, 29, 24, 16, 37, 37, 34, 34, 45, 47]
_LOC_SCALE = [0.5, 0.33, -0.5, 0.25, 0.0, 0.16, -0.16, 0.3, 0.0, -0.3, 0.5, -0.25]

_B, _C, _H, _W = 32, 256, 56, 56
_HW = _H * _W
_BSTRIDE = _C * _HW
_SCALE = min(_H, _W) / _IMG_SIZE
_NCTR = 12
_ROWS = 48

_RNE_BIAS = np.float32(1.5 * 2**23)


def _rne(v):
    return (v + _RNE_BIAS) - _RNE_BIAS


_mesh = plsc.VectorSubcoreMesh(core_axis_name="c", subcore_axis_name="s",
                               num_cores=2)


@functools.partial(
    pl.kernel,
    mesh=_mesh,
    out_type=[
        jax.ShapeDtypeStruct((_B, 2 * _NCTR, 128), jnp.float32),
        jax.ShapeDtypeStruct((_B, 2 * _NCTR, 128), jnp.float32),
    ],
    scratch_types=[
        pltpu.VMEM((128,), jnp.float32),
        pltpu.VMEM((32,), jnp.int32),
        pltpu.VMEM((16,), jnp.float32),
        pltpu.VMEM((32,), jnp.int32),
        pltpu.VMEM((_ROWS, 128), jnp.int32),
        pltpu.VMEM((_ROWS, 128), jnp.float32),
        pltpu.SemaphoreType.DMA,
    ],
    compiler_params=pltpu.CompilerParams(needs_layout_passes=False),
)
def _sc_gather(flat_hbm, lm_hbm, ctr_hbm, ls_hbm, enc_hbm, dec_hbm,
               lm_v, ctr_v, ls_v, base_v, idx_v, buf_v, sem):
    b = lax.axis_index("s") * 2 + lax.axis_index("c")

    pltpu.sync_copy(lm_hbm.at[pl.ds(b * 128, 128)], lm_v)
    pltpu.sync_copy(ctr_hbm, ctr_v)
    pltpu.sync_copy(ls_hbm, ls_v)

    def full(s):
        return jnp.full((16,), s, jnp.int32)

    ruler_a = plsc.load_gather(lm_v, [full(22)])
    ruler_b = plsc.load_gather(lm_v, [full(25)])
    scales = jnp.abs(ruler_a - ruler_b) * ls_v[...]

    def bases(ctr_idx):
        x = plsc.load_gather(lm_v, [ctr_idx])
        y = plsc.load_gather(lm_v, [ctr_idx + 64]) + scales
        xi = _rne(_rne(x) * _SCALE).astype(jnp.int32)
        yi = _rne(_rne(y) * _SCALE).astype(jnp.int32)
        xi = jnp.clip(xi, 0, _W - 1)
        yi = jnp.clip(yi, 0, _H - 1)
        return b * _BSTRIDE + xi * _W + yi

    base_v[pl.ds(0, 16)] = bases(ctr_v[pl.ds(0, 16)])
    base_v[pl.ds(16, 16)] = bases(ctr_v[pl.ds(16, 16)])

    lanes = lax.iota(jnp.int32, 16)

    def build(t, carry):
        i = t // 8
        q = t % 8
        ii = i % (2 * _NCTR)
        j = ii // 2 + (i // (2 * _NCTR)) * 16
        base = plsc.load_gather(base_v, [full(j)])
        c0 = (i % 2) * 128 + q * 16
        idx_v[i, pl.ds(q * 16, 16)] = base + (c0 + lanes) * _HW
        return carry

    lax.fori_loop(0, _ROWS * 8, build, 0)

    def fire(i, carry):
        pltpu.async_copy(flat_hbm.at[idx_v.at[i]], buf_v.at[i], sem)
        return carry

    lax.fori_loop(0, _ROWS, fire, 0)

    pltpu.make_async_copy(enc_hbm.at[b], buf_v.at[pl.ds(0, 24)], sem).wait()
    pltpu.make_async_copy(dec_hbm.at[b], buf_v.at[pl.ds(24, 24)], sem).wait()

    pltpu.sync_copy(buf_v.at[pl.ds(0, 24)], enc_hbm.at[b])
    pltpu.sync_copy(buf_v.at[pl.ds(24, 24)], dec_hbm.at[b])


def kernel(tensor, landmarks):
    batch, channels, h, w = tensor.shape
    flat = tensor.reshape(-1)
    lm_pad = jnp.pad(landmarks, ((0, 0), (0, 0), (0, 64 - landmarks.shape[-1])))
    lm_flat = lm_pad.reshape(-1)
    ctr = np.zeros((32,), np.int32)
    ctr[0:_NCTR] = _CENTERS_LEFT
    ctr[16:16 + _NCTR] = _CENTERS_RIGHT
    ls = np.zeros((16,), np.float32)
    ls[0:_NCTR] = _LOC_SCALE
    enc, dec = _sc_gather(flat, lm_flat, ctr, ls)
    return (enc.reshape(batch, _NCTR, channels),
            dec.reshape(batch, _NCTR, channels))


# speedup vs baseline: 10.8680x; 10.8680x over previous
import functools

import jax
import jax.numpy as jnp
import numpy as np
from jax import lax
from jax.experimental import pallas as pl
from jax.experimental.pallas import tpu as pltpu
from jax.experimental.pallas import tpu_sc as plsc

_IMG_SIZE = 224
_CENTERS_LEFT = [4, 1, 2, 24, 19, 16, 31, 31, 34, 34, 37, 43]
_CENTERS_RIGHT = [5, 8, 7, 29, 24, 16, 37, 37, 34, 34, 45, 47]
_LOC_SCALE = [0.5, 0.33, -0.5, 0.25, 0.0, 0.16, -0.16, 0.3, 0.0, -0.3, 0.5, -0.25]

_B, _C, _H, _W = 32, 256, 56, 56
_HW = _H * _W
_SCALE = min(_H, _W) / _IMG_SIZE
_NCTR = 12

_RNE_BIAS = np.float32(1.5 * 2**23)


def _rne(v):
    return (v + _RNE_BIAS) - _RNE_BIAS


_mesh = plsc.VectorSubcoreMesh(core_axis_name="c", subcore_axis_name="s",
                               num_cores=2)


@functools.partial(
    pl.kernel,
    mesh=_mesh,
    out_type=[
        jax.ShapeDtypeStruct((_B, 16, _C), jnp.float32),
        jax.ShapeDtypeStruct((_B, 16, _C), jnp.float32),
    ],
    scratch_types=[
        pltpu.VMEM((128,), jnp.float32),
        pltpu.VMEM((32,), jnp.int32),
        pltpu.VMEM((16,), jnp.float32),
        pltpu.VMEM((32,), jnp.int32),
        pltpu.VMEM((32, _C), jnp.float32),
        pltpu.SemaphoreType.DMA,
    ],
    compiler_params=pltpu.CompilerParams(needs_layout_passes=False),
)
def _sc_gather(rows_hbm, lm_hbm, ctr_hbm, ls_hbm, enc_hbm, dec_hbm,
               lm_v, ctr_v, ls_v, rid_v, buf_v, sem):
    b = lax.axis_index("s") * 2 + lax.axis_index("c")

    pltpu.sync_copy(lm_hbm.at[pl.ds(b * 128, 128)], lm_v)
    pltpu.sync_copy(ctr_hbm, ctr_v)
    pltpu.sync_copy(ls_hbm, ls_v)

    def full(s):
        return jnp.full((16,), s, jnp.int32)

    ruler_a = plsc.load_gather(lm_v, [full(22)])
    ruler_b = plsc.load_gather(lm_v, [full(25)])
    scales = jnp.abs(ruler_a - ruler_b) * ls_v[...]

    def row_ids(ctr_idx):
        x = plsc.load_gather(lm_v, [ctr_idx])
        y = plsc.load_gather(lm_v, [ctr_idx + 64]) + scales
        xi = _rne(_rne(x) * _SCALE).astype(jnp.int32)
        yi = _rne(_rne(y) * _SCALE).astype(jnp.int32)
        xi = jnp.clip(xi, 0, _W - 1)
        yi = jnp.clip(yi, 0, _H - 1)
        return b * _HW + xi * _W + yi

    rid_v[pl.ds(0, 16)] = row_ids(ctr_v[pl.ds(0, 16)])
    rid_v[pl.ds(16, 16)] = row_ids(ctr_v[pl.ds(16, 16)])

    pltpu.async_copy(rows_hbm.at[rid_v], buf_v, sem).wait()

    pltpu.sync_copy(buf_v.at[pl.ds(0, 16)], enc_hbm.at[b])
    pltpu.sync_copy(buf_v.at[pl.ds(16, 16)], dec_hbm.at[b])


def kernel(tensor, landmarks):
    batch, channels, h, w = tensor.shape
    rows = tensor.transpose(0, 2, 3, 1).reshape(batch * h * w, channels)
    lm_pad = jnp.pad(landmarks, ((0, 0), (0, 0), (0, 64 - landmarks.shape[-1])))
    lm_flat = lm_pad.reshape(-1)
    ctr = np.zeros((32,), np.int32)
    ctr[0:_NCTR] = _CENTERS_LEFT
    ctr[16:16 + _NCTR] = _CENTERS_RIGHT
    ls = np.zeros((16,), np.float32)
    ls[0:_NCTR] = _LOC_SCALE
    enc, dec = _sc_gather(rows, lm_flat, ctr, ls)
    return enc[:, :_NCTR, :], dec[:, :_NCTR, :]


# speedup vs baseline: 13.1077x; 1.2061x over previous
import functools

import jax
import jax.numpy as jnp
import numpy as np
from jax import lax
from jax.experimental import pallas as pl
from jax.experimental.pallas import tpu as pltpu
from jax.experimental.pallas import tpu_sc as plsc

_IMG_SIZE = 224
_CENTERS_LEFT = [4, 1, 2, 24, 19, 16, 31, 31, 34, 34, 37, 43]
_CENTERS_RIGHT = [5, 8, 7, 29, 24, 16, 37, 37, 34, 34, 45, 47]
_LOC_SCALE = [0.5, 0.33, -0.5, 0.25, 0.0, 0.16, -0.16, 0.3, 0.0, -0.3, 0.5, -0.25]

_B, _C, _H, _W = 32, 256, 56, 56
_HW = _H * _W
_SCALE = min(_H, _W) / _IMG_SIZE
_NCTR = 12

_RNE_BIAS = np.float32(1.5 * 2**23)


def _rne(v):
    return (v + _RNE_BIAS) - _RNE_BIAS


_mesh = plsc.VectorSubcoreMesh(core_axis_name="c", subcore_axis_name="s",
                               num_cores=2)


@functools.partial(
    pl.kernel,
    mesh=_mesh,
    out_type=jax.ShapeDtypeStruct((2 * _NCTR, _B, _C), jnp.float32),
    scratch_types=[
        pltpu.VMEM((4096,), jnp.float32),
        pltpu.VMEM((32,), jnp.int32),
        pltpu.VMEM((_B, _C), jnp.float32),
        pltpu.SemaphoreType.DMA,
    ],
    compiler_params=pltpu.CompilerParams(needs_layout_passes=False),
)
def _sc_gather(rows_hbm, lm_hbm, out_hbm, lm_v, rid_v, buf_v, sem):
    w = lax.axis_index("s") * 2 + lax.axis_index("c")

    @pl.when(w < 2 * _NCTR)
    def _body():
        pltpu.sync_copy(lm_hbm, lm_v)

        is_right = (w >= _NCTR).astype(jnp.int32)
        j = w - _NCTR * is_right
        cid_s = jnp.int32(0)
        lsj_s = jnp.float32(0)
        for i in range(_NCTR):
            m = (j == i).astype(jnp.int32)
            cid_s = cid_s + m * (_CENTERS_LEFT[i]
                                 + is_right * (_CENTERS_RIGHT[i] - _CENTERS_LEFT[i]))
            lsj_s = lsj_s + m.astype(jnp.float32) * _LOC_SCALE[i]
        cid = jnp.full((16,), cid_s, jnp.int32)
        lsj = jnp.full((16,), lsj_s, jnp.float32)

        lanes = lax.iota(jnp.int32, 16)
        for k in range(2):
            base = (k * 16 + lanes) * 128
            xr = plsc.load_gather(lm_v, [base + 22])
            yr = plsc.load_gather(lm_v, [base + 25])
            scale = jnp.abs(xr - yr) * lsj
            x = plsc.load_gather(lm_v, [base + cid])
            y = plsc.load_gather(lm_v, [base + 64 + cid]) + scale
            xi = _rne(_rne(x) * _SCALE).astype(jnp.int32)
            yi = _rne(_rne(y) * _SCALE).astype(jnp.int32)
            xi = jnp.clip(xi, 0, _W - 1)
            yi = jnp.clip(yi, 0, _H - 1)
            rid_v[pl.ds(k * 16, 16)] = (k * 16 + lanes) * _HW + xi * _W + yi

        pltpu.async_copy(rows_hbm.at[rid_v], buf_v, sem).wait()
        pltpu.sync_copy(buf_v, out_hbm.at[w])


def kernel(tensor, landmarks):
    batch, channels, h, w = tensor.shape
    rows = tensor.transpose(0, 2, 3, 1).reshape(batch * h * w, channels)
    lm_pad = jnp.pad(landmarks, ((0, 0), (0, 0), (0, 64 - landmarks.shape[-1])))
    lm_flat = lm_pad.reshape(-1)
    out = _sc_gather(rows, lm_flat)
    return (out[:_NCTR].transpose(1, 0, 2), out[_NCTR:].transpose(1, 0, 2))


# speedup vs baseline: 14.0985x; 1.0756x over previous
import functools

import jax
import jax.numpy as jnp
import numpy as np
from jax import lax
from jax.experimental import pallas as pl
from jax.experimental.pallas import tpu as pltpu
from jax.experimental.pallas import tpu_sc as plsc

_IMG_SIZE = 224
_CENTERS_LEFT = [4, 1, 2, 24, 19, 16, 31, 31, 34, 34, 37, 43]
_CENTERS_RIGHT = [5, 8, 7, 29, 24, 16, 37, 37, 34, 34, 45, 47]
_LOC_SCALE = [0.5, 0.33, -0.5, 0.25, 0.0, 0.16, -0.16, 0.3, 0.0, -0.3, 0.5, -0.25]

_B, _C, _H, _W = 32, 256, 56, 56
_HW = _H * _W
_SCALE = min(_H, _W) / _IMG_SIZE
_NCTR = 12
_NLM = 49

_RNE_BIAS = np.float32(1.5 * 2**23)


def _rne(v):
    return (v + _RNE_BIAS) - _RNE_BIAS


_mesh = plsc.VectorSubcoreMesh(core_axis_name="c", subcore_axis_name="s",
                               num_cores=2)


@functools.partial(
    pl.kernel,
    mesh=_mesh,
    out_type=[
        jax.ShapeDtypeStruct((_NCTR, _B, _C), jnp.float32),
        jax.ShapeDtypeStruct((_NCTR, _B, _C), jnp.float32),
    ],
    scratch_types=[
        pltpu.VMEM((_B, 2, _NLM), jnp.float32),
        pltpu.VMEM((32,), jnp.int32),
        pltpu.VMEM((_B, _C), jnp.float32),
        pltpu.SemaphoreType.DMA,
        pltpu.SemaphoreType.DMA,
    ],
    compiler_params=pltpu.CompilerParams(needs_layout_passes=False),
)
def _sc_gather(rows_hbm, lm_hbm, enc_hbm, dec_hbm, lm_v, rid_v, buf_v,
               lm_sem, sem):
    w = lax.axis_index("s") * 2 + lax.axis_index("c")

    @pl.when(w < 2 * _NCTR)
    def _body():
        lm_copy = pltpu.async_copy(lm_hbm, lm_v, lm_sem)

        is_right = (w >= _NCTR).astype(jnp.int32)
        j = w - _NCTR * is_right
        cid_s = jnp.int32(0)
        lsj_s = jnp.float32(0)
        for i in range(_NCTR):
            m = (j == i).astype(jnp.int32)
            cid_s = cid_s + m * (_CENTERS_LEFT[i]
                                 + is_right * (_CENTERS_RIGHT[i] - _CENTERS_LEFT[i]))
            lsj_s = lsj_s + m.astype(jnp.float32) * _LOC_SCALE[i]
        cid = jnp.full((16,), cid_s, jnp.int32)
        lsj = jnp.full((16,), lsj_s, jnp.float32)

        lm_copy.wait()
        lanes = lax.iota(jnp.int32, 16)
        zero = jnp.zeros((16,), jnp.int32)
        one = jnp.full((16,), 1, jnp.int32)
        for k in range(2):
            b_vec = k * 16 + lanes
            xr = plsc.load_gather(lm_v, [b_vec, zero, jnp.full((16,), 22, jnp.int32)])
            yr = plsc.load_gather(lm_v, [b_vec, zero, jnp.full((16,), 25, jnp.int32)])
            scale = jnp.abs(xr - yr) * lsj
            x = plsc.load_gather(lm_v, [b_vec, zero, cid])
            y = plsc.load_gather(lm_v, [b_vec, one, cid]) + scale
            xi = _rne(_rne(x) * _SCALE).astype(jnp.int32)
            yi = _rne(_rne(y) * _SCALE).astype(jnp.int32)
            xi = jnp.clip(xi, 0, _W - 1)
            yi = jnp.clip(yi, 0, _H - 1)
            rid_v[pl.ds(k * 16, 16)] = b_vec * _HW + xi * _W + yi

        pltpu.async_copy(rows_hbm.at[rid_v], buf_v, sem).wait()

        @pl.when(is_right == 0)
        def _enc():
            pltpu.sync_copy(buf_v, enc_hbm.at[j])

        @pl.when(is_right == 1)
        def _dec():
            pltpu.sync_copy(buf_v, dec_hbm.at[j])


def kernel(tensor, landmarks):
    batch, channels, h, w = tensor.shape
    rows = tensor.transpose(0, 2, 3, 1).reshape(batch * h * w, channels)
    enc, dec = _sc_gather(rows, landmarks)
    return (enc.transpose(1, 0, 2), dec.transpose(1, 0, 2))


# speedup vs baseline: 14.1022x; 1.0003x over previous
import functools

import jax
import jax.numpy as jnp
import numpy as np
from jax import lax
from jax.experimental import pallas as pl
from jax.experimental.pallas import tpu as pltpu
from jax.experimental.pallas import tpu_sc as plsc

_IMG_SIZE = 224
_CENTERS_LEFT = [4, 1, 2, 24, 19, 16, 31, 31, 34, 34, 37, 43]
_CENTERS_RIGHT = [5, 8, 7, 29, 24, 16, 37, 37, 34, 34, 45, 47]
_LOC_SCALE = [0.5, 0.33, -0.5, 0.25, 0.0, 0.16, -0.16, 0.3, 0.0, -0.3, 0.5, -0.25]

_B, _C, _H, _W = 32, 256, 56, 56
_HW = _H * _W
_SCALE = min(_H, _W) / _IMG_SIZE
_NCTR = 12
_NLM = 49

_RNE_BIAS = np.float32(1.5 * 2**23)


def _rne(v):
    return (v + _RNE_BIAS) - _RNE_BIAS


_mesh = plsc.VectorSubcoreMesh(core_axis_name="c", subcore_axis_name="s",
                               num_cores=2)


@functools.partial(
    pl.kernel,
    mesh=_mesh,
    out_type=[
        jax.ShapeDtypeStruct((_NCTR, _B, _C), jnp.float32),
        jax.ShapeDtypeStruct((_NCTR, _B, _C), jnp.float32),
    ],
    scratch_types=[
        pltpu.VMEM((_B, 2, _NLM), jnp.float32),
        pltpu.VMEM((32,), jnp.int32),
        pltpu.VMEM((_B, _C), jnp.float32),
        pltpu.SemaphoreType.DMA,
        pltpu.SemaphoreType.DMA,
    ],
    compiler_params=pltpu.CompilerParams(needs_layout_passes=False, skip_device_barrier=True, disable_bounds_checks=True, disable_semaphore_checks=True),
)
def _sc_gather(rows_hbm, lm_hbm, enc_hbm, dec_hbm, lm_v, rid_v, buf_v,
               lm_sem, sem):
    w = lax.axis_index("s") * 2 + lax.axis_index("c")

    @pl.when(w < 2 * _NCTR)
    def _body():
        lm_copy = pltpu.async_copy(lm_hbm, lm_v, lm_sem)

        is_right = (w >= _NCTR).astype(jnp.int32)
        j = w - _NCTR * is_right
        cid_s = jnp.int32(0)
        lsj_s = jnp.float32(0)
        for i in range(_NCTR):
            m = (j == i).astype(jnp.int32)
            cid_s = cid_s + m * (_CENTERS_LEFT[i]
                                 + is_right * (_CENTERS_RIGHT[i] - _CENTERS_LEFT[i]))
            lsj_s = lsj_s + m.astype(jnp.float32) * _LOC_SCALE[i]
        cid = jnp.full((16,), cid_s, jnp.int32)
        lsj = jnp.full((16,), lsj_s, jnp.float32)

        lm_copy.wait()
        lanes = lax.iota(jnp.int32, 16)
        zero = jnp.zeros((16,), jnp.int32)
        one = jnp.full((16,), 1, jnp.int32)
        for k in range(2):
            b_vec = k * 16 + lanes
            xr = plsc.load_gather(lm_v, [b_vec, zero, jnp.full((16,), 22, jnp.int32)])
            yr = plsc.load_gather(lm_v, [b_vec, zero, jnp.full((16,), 25, jnp.int32)])
            scale = jnp.abs(xr - yr) * lsj
            x = plsc.load_gather(lm_v, [b_vec, zero, cid])
            y = plsc.load_gather(lm_v, [b_vec, one, cid]) + scale
            xi = _rne(_rne(x) * _SCALE).astype(jnp.int32)
            yi = _rne(_rne(y) * _SCALE).astype(jnp.int32)
            xi = jnp.clip(xi, 0, _W - 1)
            yi = jnp.clip(yi, 0, _H - 1)
            rid_v[pl.ds(k * 16, 16)] = b_vec * _HW + xi * _W + yi

        pltpu.async_copy(rows_hbm.at[rid_v], buf_v, sem).wait()

        @pl.when(is_right == 0)
        def _enc():
            pltpu.sync_copy(buf_v, enc_hbm.at[j])

        @pl.when(is_right == 1)
        def _dec():
            pltpu.sync_copy(buf_v, dec_hbm.at[j])


def kernel(tensor, landmarks):
    batch, channels, h, w = tensor.shape
    rows = tensor.transpose(0, 2, 3, 1).reshape(batch * h * w, channels)
    enc, dec = _sc_gather(rows, landmarks)
    return (enc.transpose(1, 0, 2), dec.transpose(1, 0, 2))
